# baseline (speedup 1.0000x reference)
import functools

import jax
import jax.numpy as jnp
from jax import lax
from jax.experimental import pallas as pl
from jax.experimental.pallas import tpu as pltpu
from jax.experimental.pallas import tpu_sc as plsc

_N, _E, _D, _FD = 10000, 320000, 128, 64
_DP = 144
_L = 16
_NC, _NS = 2, 16
_NW = _NC * _NS
_EPW = _E // _NW
_C = 40
_NCHUNK = _EPW // _C
_NP = 10240
_RPT = _NP // _NS


_BM = 2000

_GDN = lax.GatherDimensionNumbers(
    offset_dims=(), collapsed_slice_dims=(0,), start_index_map=(0,)
)


def _rot(v, r):
    perm = ((jnp.arange(_L, dtype=jnp.int32) + r) % _L)[:, None]
    return lax.gather(v, perm, dimension_numbers=_GDN, slice_sizes=(1,),
                      mode=lax.GatherScatterMode.PROMISE_IN_BOUNDS)


def _lane_sum(v):
    for r in (8, 4, 2, 1):
        v = v + _rot(v, r)
    return v


def _mm_body(x_ref, feat_ref, wfc_ref, wdst_ref, z_ref, f_ref):
    z = jnp.dot(x_ref[...], wfc_ref[...], preferred_element_type=jnp.float32)
    z_ref[:, : _D] = z
    col = lax.broadcasted_iota(jnp.int32, (_BM, _DP - _D), 1)
    z_ref[:, _D:] = jnp.where(col == 0, 1.0, 0.0).astype(jnp.float32)
    f_ref[...] = jnp.dot(
        feat_ref[...], wdst_ref[...], preferred_element_type=jnp.float32
    )


def _matmuls(x, feat, w_fc, w_dst):
    grid = (_N // _BM,)
    return pl.pallas_call(
        _mm_body,
        grid=grid,
        in_specs=[
            pl.BlockSpec((_BM, _D), lambda i: (i, 0)),
            pl.BlockSpec((_BM, _FD), lambda i: (i, 0)),
            pl.BlockSpec((_D, _D), lambda i: (0, 0)),
            pl.BlockSpec((_FD, _D), lambda i: (0, 0)),
        ],
        out_specs=[
            pl.BlockSpec((_BM, _DP), lambda i: (i, 0)),
            pl.BlockSpec((_BM, _D), lambda i: (i, 0)),
        ],
        out_shape=[
            jax.ShapeDtypeStruct((_N, _DP), jnp.float32),
            jax.ShapeDtypeStruct((_N, _D), jnp.float32),
        ],
    )(x, feat, w_fc, w_dst)


def _edge_body(z_hbm, f_hbm, src_hbm, dst_hbm, out_hbm,
               acc, sidx, didx, sdix, zrows, frows, srows,
               semz, semf, semi, sems):
    cid = lax.axis_index("c")
    sid = lax.axis_index("s")

    zero = jnp.zeros((_L,), jnp.float32)
    for j in range(_C):
        for k in range(_DP // _L):
            srows[0][j, pl.ds(k * _L, _L)] = zero
    row0 = sid * _RPT
    for t in range(_RPT // _C):
        pltpu.sync_copy(srows[0], acc.at[pl.ds(row0 + t * _C, _C)])
    plsc.subcore_barrier()

    wid = sid * _NC + cid
    base = wid * _EPW

    def idx_copies(c, p):
        off = base + c * _C
        return (
            pltpu.make_async_copy(src_hbm.at[pl.ds(off, _C)], sidx[p], semi),
            pltpu.make_async_copy(dst_hbm.at[pl.ds(off, _C)], didx[p], semi),
        )

    def gathers(c, p):
        return (
            pltpu.make_async_copy(z_hbm.at[sidx[p]], zrows[p], semz),
            pltpu.make_async_copy(f_hbm.at[didx[p]], frows[p], semf),
        )

    i0s, i0d = idx_copies(0, 0)
    i0s.start()
    i0d.start()
    i1s, i1d = idx_copies(1, 1)
    i1s.start()
    i1d.start()
    i0s.wait()
    i0d.wait()
    g0z, g0f = gathers(0, 0)
    g0z.start()
    g0f.start()

    def pair(i, carry):
        for p in range(2):
            c = 2 * i + p
            gz, gf = gathers(c, p)
            gz.wait()
            gf.wait()

            @pl.when(c + 1 < _NCHUNK)
            def _issue_next():
                ws, wd = idx_copies(c + 1, 1 - p)
                ws.wait()
                wd.wait()
                nz, nf = gathers(c + 1, 1 - p)
                nz.start()
                nf.start()

            @pl.when(c >= 2)
            def _drain():
                pltpu.make_async_copy(srows[p], acc.at[sdix[p]],
                                      sems[p]).wait()

            for off in (0, _L, _C - _L):
                sdix[p][pl.ds(off, _L)] = didx[p][pl.ds(off, _L)]

            @pl.when(c + 2 < _NCHUNK)
            def _prefetch_idx():
                ns, nd = idx_copies(c + 2, p)
                ns.start()
                nd.start()

            @plsc.parallel_loop(0, _C, 1, unroll=8)
            def edge(j):
                accv = zrows[p][j, pl.ds(0, _L)] * frows[p][j, pl.ds(0, _L)]
                for k in range(1, _D // _L):
                    accv = accv + (zrows[p][j, pl.ds(k * _L, _L)]
                                   * frows[p][j, pl.ds(k * _L, _L)])
                ex = jnp.exp(_lane_sum(accv))
                for k in range(_DP // _L):
                    srows[p][j, pl.ds(k * _L, _L)] = (
                        zrows[p][j, pl.ds(k * _L, _L)] * ex)

            pltpu.async_copy(srows[p], acc.at[sdix[p]], sems[p], add=True)
        return carry

    lax.fori_loop(0, _NCHUNK // 2, pair, 0)
    pltpu.make_async_copy(srows[0], acc.at[sdix[0]], sems[0]).wait()
    pltpu.make_async_copy(srows[1], acc.at[sdix[1]], sems[1]).wait()
    plsc.subcore_barrier()

    pltpu.sync_copy(acc.at[pl.ds(row0, _RPT)],
                    out_hbm.at[cid, pl.ds(row0, _RPT)])


def _edge_pass(z_pad, dstf, src, dst):
    mesh = plsc.VectorSubcoreMesh(core_axis_name="c", subcore_axis_name="s")
    k = pl.kernel(
        lambda z, f, s, d, o, acc, s0, s1, d0, d1, x0, x1, z0, z1, f0, f1,
               sr0, sr1, sz, sf, si, ss0, ss1: _edge_body(
            z, f, s, d, o, acc, (s0, s1), (d0, d1), (x0, x1), (z0, z1),
            (f0, f1), (sr0, sr1), sz, sf, si, (ss0, ss1)),
        out_type=jax.ShapeDtypeStruct((_NC, _NP, _DP), jnp.float32),
        mesh=mesh,
        scratch_types=[
            pltpu.VMEM_SHARED((_NP, _DP), jnp.float32),
            pltpu.VMEM((_C,), jnp.int32),
            pltpu.VMEM((_C,), jnp.int32),
            pltpu.VMEM((_C,), jnp.int32),
            pltpu.VMEM((_C,), jnp.int32),
            pltpu.VMEM((_C,), jnp.int32),
            pltpu.VMEM((_C,), jnp.int32),
            pltpu.VMEM((_C, _DP), jnp.float32),
            pltpu.VMEM((_C, _DP), jnp.float32),
            pltpu.VMEM((_C, _D), jnp.float32),
            pltpu.VMEM((_C, _D), jnp.float32),
            pltpu.VMEM((_C, _DP), jnp.float32),
            pltpu.VMEM((_C, _DP), jnp.float32),
            pltpu.SemaphoreType.DMA,
            pltpu.SemaphoreType.DMA,
            pltpu.SemaphoreType.DMA,
            pltpu.SemaphoreType.DMA,
            pltpu.SemaphoreType.DMA,
        ],
        compiler_params=pltpu.CompilerParams(use_tc_tiling_on_sc=False),
    )
    return k(z_pad, dstf, src, dst)


_BC = 2000


def _combine_body(p_ref, o_ref):
    s = p_ref[0] + p_ref[1]
    num = s[:, : _D]
    den = s[:, _D : _D + 1]
    o_ref[...] = jnp.where(den > 0.0, num / den, 0.0)


def _combine(partials):
    grid = (_N // _BC,)
    return pl.pallas_call(
        _combine_body,
        grid=grid,
        in_specs=[pl.BlockSpec((_NC, _BC, _DP), lambda i: (0, i, 0))],
        out_specs=pl.BlockSpec((_BC, _D), lambda i: (i, 0)),
        out_shape=jax.ShapeDtypeStruct((_N, _D), jnp.float32),
    )(partials)


def kernel(x, feat, edge_index, W_fc, W_dst):
    src = edge_index[0]
    dst = edge_index[1]
    z_pad, dstf = _matmuls(x, feat, W_fc, W_dst)
    partials = _edge_pass(z_pad, dstf, src, dst)
    s = partials[0, : _N] + partials[1, : _N]
    den = s[:, _D : _D + 1]
    return jnp.where(den > 0.0, s[:, : _D] / den, 0.0)


# speedup vs baseline: 1.0800x; 1.0800x over previous
import functools

import jax
import jax.numpy as jnp
from jax import lax
from jax.experimental import pallas as pl
from jax.experimental.pallas import tpu as pltpu
from jax.experimental.pallas import tpu_sc as plsc

_N, _E, _D, _FD = 10000, 320000, 128, 64
_DP = 144
_L = 16
_NC, _NS = 2, 16
_NW = _NC * _NS
_EPW = _E // _NW
_C = 40
_NCHUNK = _EPW // _C
_NP = 10240
_RPT = _NP // _NS


_BM = 2000

_GDN = lax.GatherDimensionNumbers(
    offset_dims=(), collapsed_slice_dims=(0,), start_index_map=(0,)
)


def _rot(v, r):
    perm = ((jnp.arange(_L, dtype=jnp.int32) + r) % _L)[:, None]
    return lax.gather(v, perm, dimension_numbers=_GDN, slice_sizes=(1,),
                      mode=lax.GatherScatterMode.PROMISE_IN_BOUNDS)


def _lane_sum(v):
    for r in (8, 4, 2, 1):
        v = v + _rot(v, r)
    return v


def _mm_body(x_ref, feat_ref, wfc_ref, wdst_ref, z_ref, f_ref):
    z_ref[...] = jnp.dot(
        x_ref[...], wfc_ref[...], preferred_element_type=jnp.float32
    )
    f_ref[...] = jnp.dot(
        feat_ref[...], wdst_ref[...], preferred_element_type=jnp.float32
    )


def _matmuls(x, feat, w_fc, w_dst):
    grid = (_N // _BM,)
    return pl.pallas_call(
        _mm_body,
        grid=grid,
        in_specs=[
            pl.BlockSpec((_BM, _D), lambda i: (i, 0)),
            pl.BlockSpec((_BM, _FD), lambda i: (i, 0)),
            pl.BlockSpec((_D, _D), lambda i: (0, 0)),
            pl.BlockSpec((_FD, _D), lambda i: (0, 0)),
        ],
        out_specs=[
            pl.BlockSpec((_BM, _D), lambda i: (i, 0)),
            pl.BlockSpec((_BM, _D), lambda i: (i, 0)),
        ],
        out_shape=[
            jax.ShapeDtypeStruct((_N, _D), jnp.float32),
            jax.ShapeDtypeStruct((_N, _D), jnp.float32),
        ],
    )(x, feat, w_fc, w_dst)


def _edge_body(z_hbm, f_hbm, src_hbm, dst_hbm, out_hbm,
               acc, sidx, didx, sdix, zrows, frows, srows,
               semz, semf, semi, sems):
    cid = lax.axis_index("c")
    sid = lax.axis_index("s")

    zero = jnp.zeros((_L,), jnp.float32)
    for j in range(_C):
        for k in range(_DP // _L):
            srows[0][j, pl.ds(k * _L, _L)] = zero
    row0 = sid * _RPT
    for t in range(_RPT // _C):
        pltpu.async_copy(srows[0], acc.at[pl.ds(row0 + t * _C, _C)], semz)
    for t in range(_RPT // _C):
        pltpu.make_async_copy(srows[0], acc.at[pl.ds(row0 + t * _C, _C)],
                              semz).wait()
    plsc.subcore_barrier()

    wid = sid * _NC + cid
    base = wid * _EPW

    def idx_copies(c, p):
        off = base + c * _C
        return (
            pltpu.make_async_copy(src_hbm.at[pl.ds(off, _C)], sidx[p], semi),
            pltpu.make_async_copy(dst_hbm.at[pl.ds(off, _C)], didx[p], semi),
        )

    def gathers(c, p):
        return (
            pltpu.make_async_copy(z_hbm.at[sidx[p]], zrows[p], semz),
            pltpu.make_async_copy(f_hbm.at[didx[p]], frows[p], semf),
        )

    i0s, i0d = idx_copies(0, 0)
    i0s.start()
    i0d.start()
    i1s, i1d = idx_copies(1, 1)
    i1s.start()
    i1d.start()
    i0s.wait()
    i0d.wait()
    g0z, g0f = gathers(0, 0)
    g0z.start()
    g0f.start()

    def pair(i, carry):
        for p in range(2):
            c = 2 * i + p
            gz, gf = gathers(c, p)
            gz.wait()
            gf.wait()

            @pl.when(c + 1 < _NCHUNK)
            def _issue_next():
                ws, wd = idx_copies(c + 1, 1 - p)
                ws.wait()
                wd.wait()
                nz, nf = gathers(c + 1, 1 - p)
                nz.start()
                nf.start()

            @pl.when(c >= 2)
            def _drain():
                pltpu.make_async_copy(srows[p], acc.at[sdix[p]],
                                      sems[p]).wait()

            for off in (0, _L, _C - _L):
                sdix[p][pl.ds(off, _L)] = didx[p][pl.ds(off, _L)]

            @pl.when(c + 2 < _NCHUNK)
            def _prefetch_idx():
                ns, nd = idx_copies(c + 2, p)
                ns.start()
                nd.start()

            lane0 = jnp.where(
                lax.broadcasted_iota(jnp.int32, (_L,), 0) == 0, 1.0, 0.0
            ).astype(jnp.float32)

            @plsc.parallel_loop(0, _C, 1, unroll=8)
            def edge(j):
                accv = zrows[p][j, pl.ds(0, _L)] * frows[p][j, pl.ds(0, _L)]
                for k in range(1, _D // _L):
                    accv = accv + (zrows[p][j, pl.ds(k * _L, _L)]
                                   * frows[p][j, pl.ds(k * _L, _L)])
                ex = jnp.exp(_lane_sum(accv))
                for k in range(_D // _L):
                    srows[p][j, pl.ds(k * _L, _L)] = (
                        zrows[p][j, pl.ds(k * _L, _L)] * ex)
                srows[p][j, pl.ds(_D, _L)] = ex * lane0

            pltpu.async_copy(srows[p], acc.at[sdix[p]], sems[p], add=True)
        return carry

    lax.fori_loop(0, _NCHUNK // 2, pair, 0)
    pltpu.make_async_copy(srows[0], acc.at[sdix[0]], sems[0]).wait()
    pltpu.make_async_copy(srows[1], acc.at[sdix[1]], sems[1]).wait()
    plsc.subcore_barrier()

    pltpu.sync_copy(acc.at[pl.ds(row0, _RPT)],
                    out_hbm.at[cid, pl.ds(row0, _RPT)])


def _edge_pass(z_pad, dstf, src, dst):
    mesh = plsc.VectorSubcoreMesh(core_axis_name="c", subcore_axis_name="s")
    k = pl.kernel(
        lambda z, f, s, d, o, acc, s0, s1, d0, d1, x0, x1, z0, z1, f0, f1,
               sr0, sr1, sz, sf, si, ss0, ss1: _edge_body(
            z, f, s, d, o, acc, (s0, s1), (d0, d1), (x0, x1), (z0, z1),
            (f0, f1), (sr0, sr1), sz, sf, si, (ss0, ss1)),
        out_type=jax.ShapeDtypeStruct((_NC, _NP, _DP), jnp.float32),
        mesh=mesh,
        scratch_types=[
            pltpu.VMEM_SHARED((_NP, _DP), jnp.float32),
            pltpu.VMEM((_C,), jnp.int32),
            pltpu.VMEM((_C,), jnp.int32),
            pltpu.VMEM((_C,), jnp.int32),
            pltpu.VMEM((_C,), jnp.int32),
            pltpu.VMEM((_C,), jnp.int32),
            pltpu.VMEM((_C,), jnp.int32),
            pltpu.VMEM((_C, _D), jnp.float32),
            pltpu.VMEM((_C, _D), jnp.float32),
            pltpu.VMEM((_C, _D), jnp.float32),
            pltpu.VMEM((_C, _D), jnp.float32),
            pltpu.VMEM((_C, _DP), jnp.float32),
            pltpu.VMEM((_C, _DP), jnp.float32),
            pltpu.SemaphoreType.DMA,
            pltpu.SemaphoreType.DMA,
            pltpu.SemaphoreType.DMA,
            pltpu.SemaphoreType.DMA,
            pltpu.SemaphoreType.DMA,
        ],
        compiler_params=pltpu.CompilerParams(use_tc_tiling_on_sc=False),
    )
    return k(z_pad, dstf, src, dst)


_BC = 2000


def _combine_body(p_ref, o_ref):
    s = p_ref[0] + p_ref[1]
    num = s[:, : _D]
    den = s[:, _D : _D + 1]
    o_ref[...] = jnp.where(den > 0.0, num / den, 0.0)


def _combine(partials):
    grid = (_N // _BC,)
    return pl.pallas_call(
        _combine_body,
        grid=grid,
        in_specs=[pl.BlockSpec((_NC, _BC, _DP), lambda i: (0, i, 0))],
        out_specs=pl.BlockSpec((_BC, _D), lambda i: (i, 0)),
        out_shape=jax.ShapeDtypeStruct((_N, _D), jnp.float32),
    )(partials)


def kernel(x, feat, edge_index, W_fc, W_dst):
    src = edge_index[0]
    dst = edge_index[1]
    z_pad, dstf = _matmuls(x, feat, W_fc, W_dst)
    partials = _edge_pass(z_pad, dstf, src, dst)
    return _combine(partials)


# speedup vs baseline: 1.1168x; 1.0341x over previous
import functools

import jax
import jax.numpy as jnp
from jax import lax
from jax.experimental import pallas as pl
from jax.experimental.pallas import tpu as pltpu
from jax.experimental.pallas import tpu_sc as plsc

_N, _E, _D, _FD = 10000, 320000, 128, 64
_DP = 144
_L = 16
_NC, _NS = 2, 16
_NW = _NC * _NS
_EPW = _E // _NW
_C = 40
_NCHUNK = _EPW // _C
_NP = 10240
_RPT = _NP // _NS


_BM = 2000

_GDN = lax.GatherDimensionNumbers(
    offset_dims=(), collapsed_slice_dims=(0,), start_index_map=(0,)
)


def _rot(v, r):
    perm = ((jnp.arange(_L, dtype=jnp.int32) + r) % _L)[:, None]
    return lax.gather(v, perm, dimension_numbers=_GDN, slice_sizes=(1,),
                      mode=lax.GatherScatterMode.PROMISE_IN_BOUNDS)


def _lane_sum(v):
    for r in (8, 4, 2, 1):
        v = v + _rot(v, r)
    return v


def _mm_body(x_ref, feat_ref, wfc_ref, wdst_ref, z_ref, f_ref):
    z_ref[...] = jnp.dot(
        x_ref[...], wfc_ref[...], preferred_element_type=jnp.float32
    )
    f_ref[...] = jnp.dot(
        feat_ref[...], wdst_ref[...], preferred_element_type=jnp.float32
    )


def _matmuls(x, feat, w_fc, w_dst):
    grid = (_N // _BM,)
    return pl.pallas_call(
        _mm_body,
        grid=grid,
        in_specs=[
            pl.BlockSpec((_BM, _D), lambda i: (i, 0)),
            pl.BlockSpec((_BM, _FD), lambda i: (i, 0)),
            pl.BlockSpec((_D, _D), lambda i: (0, 0)),
            pl.BlockSpec((_FD, _D), lambda i: (0, 0)),
        ],
        out_specs=[
            pl.BlockSpec((_BM, _D), lambda i: (i, 0)),
            pl.BlockSpec((_BM, _D), lambda i: (i, 0)),
        ],
        out_shape=[
            jax.ShapeDtypeStruct((_N, _D), jnp.float32),
            jax.ShapeDtypeStruct((_N, _D), jnp.float32),
        ],
    )(x, feat, w_fc, w_dst)


def _edge_body(z_hbm, f_hbm, ei_hbm, out_hbm,
               acc, sidx, didx, sdix, zrows, frows, srows,
               semz, semf, semi, sems):
    cid = lax.axis_index("c")
    sid = lax.axis_index("s")

    zero = jnp.zeros((_L,), jnp.float32)
    for j in range(_C):
        for k in range(_DP // _L):
            srows[0][j, pl.ds(k * _L, _L)] = zero
    row0 = sid * _RPT
    for t in range(_RPT // _C):
        pltpu.async_copy(srows[0], acc.at[pl.ds(row0 + t * _C, _C)], semz)
    for t in range(_RPT // _C):
        pltpu.make_async_copy(srows[0], acc.at[pl.ds(row0 + t * _C, _C)],
                              semz).wait()
    plsc.subcore_barrier()

    wid = sid * _NC + cid
    base = wid * _EPW

    def idx_copies(c, p):
        off = base + c * _C
        return (
            pltpu.make_async_copy(ei_hbm.at[0, pl.ds(off, _C)], sidx[p], semi),
            pltpu.make_async_copy(ei_hbm.at[1, pl.ds(off, _C)], didx[p], semi),
        )

    def gathers(c, p):
        return (
            pltpu.make_async_copy(z_hbm.at[sidx[p]], zrows[p], semz),
            pltpu.make_async_copy(f_hbm.at[didx[p]], frows[p], semf),
        )

    i0s, i0d = idx_copies(0, 0)
    i0s.start()
    i0d.start()
    i1s, i1d = idx_copies(1, 1)
    i1s.start()
    i1d.start()
    i0s.wait()
    i0d.wait()
    g0z, g0f = gathers(0, 0)
    g0z.start()
    g0f.start()

    def pair(i, carry):
        for p in range(2):
            c = 2 * i + p
            gz, gf = gathers(c, p)
            gz.wait()
            gf.wait()

            @pl.when(c + 1 < _NCHUNK)
            def _issue_next():
                ws, wd = idx_copies(c + 1, 1 - p)
                ws.wait()
                wd.wait()
                nz, nf = gathers(c + 1, 1 - p)
                nz.start()
                nf.start()

            @pl.when(c >= 2)
            def _drain():
                pltpu.make_async_copy(srows[p], acc.at[sdix[p]],
                                      sems[p]).wait()

            for off in (0, _L, _C - _L):
                sdix[p][pl.ds(off, _L)] = didx[p][pl.ds(off, _L)]

            @pl.when(c + 2 < _NCHUNK)
            def _prefetch_idx():
                ns, nd = idx_copies(c + 2, p)
                ns.start()
                nd.start()

            lane0 = jnp.where(
                lax.broadcasted_iota(jnp.int32, (_L,), 0) == 0, 1.0, 0.0
            ).astype(jnp.float32)

            @plsc.parallel_loop(0, _C, 1, unroll=8)
            def edge(j):
                accv = zrows[p][j, pl.ds(0, _L)] * frows[p][j, pl.ds(0, _L)]
                for k in range(1, _D // _L):
                    accv = accv + (zrows[p][j, pl.ds(k * _L, _L)]
                                   * frows[p][j, pl.ds(k * _L, _L)])
                ex = jnp.exp(_lane_sum(accv))
                for k in range(_D // _L):
                    srows[p][j, pl.ds(k * _L, _L)] = (
                        zrows[p][j, pl.ds(k * _L, _L)] * ex)
                srows[p][j, pl.ds(_D, _L)] = ex * lane0

            pltpu.async_copy(srows[p], acc.at[sdix[p]], sems[p], add=True)
        return carry

    lax.fori_loop(0, _NCHUNK // 2, pair, 0)
    pltpu.make_async_copy(srows[0], acc.at[sdix[0]], sems[0]).wait()
    pltpu.make_async_copy(srows[1], acc.at[sdix[1]], sems[1]).wait()
    plsc.subcore_barrier()

    pltpu.sync_copy(acc.at[pl.ds(row0, _RPT)],
                    out_hbm.at[cid, pl.ds(row0, _RPT)])


def _edge_pass(z_pad, dstf, edge_index):
    mesh = plsc.VectorSubcoreMesh(core_axis_name="c", subcore_axis_name="s")
    k = pl.kernel(
        lambda z, f, ei, o, acc, s0, s1, d0, d1, x0, x1, z0, z1, f0, f1,
               sr0, sr1, sz, sf, si, ss0, ss1: _edge_body(
            z, f, ei, o, acc, (s0, s1), (d0, d1), (x0, x1), (z0, z1),
            (f0, f1), (sr0, sr1), sz, sf, si, (ss0, ss1)),
        out_type=jax.ShapeDtypeStruct((_NC, _NP, _DP), jnp.float32),
        mesh=mesh,
        scratch_types=[
            pltpu.VMEM_SHARED((_NP, _DP), jnp.float32),
            pltpu.VMEM((_C,), jnp.int32),
            pltpu.VMEM((_C,), jnp.int32),
            pltpu.VMEM((_C,), jnp.int32),
            pltpu.VMEM((_C,), jnp.int32),
            pltpu.VMEM((_C,), jnp.int32),
            pltpu.VMEM((_C,), jnp.int32),
            pltpu.VMEM((_C, _D), jnp.float32),
            pltpu.VMEM((_C, _D), jnp.float32),
            pltpu.VMEM((_C, _D), jnp.float32),
            pltpu.VMEM((_C, _D), jnp.float32),
            pltpu.VMEM((_C, _DP), jnp.float32),
            pltpu.VMEM((_C, _DP), jnp.float32),
            pltpu.SemaphoreType.DMA,
            pltpu.SemaphoreType.DMA,
            pltpu.SemaphoreType.DMA,
            pltpu.SemaphoreType.DMA,
            pltpu.SemaphoreType.DMA,
        ],
        compiler_params=pltpu.CompilerParams(use_tc_tiling_on_sc=False),
    )
    return k(z_pad, dstf, edge_index)


_BC = 2000


def _combine_body(p_ref, o_ref):
    s = p_ref[0] + p_ref[1]
    num = s[:, : _D]
    den = s[:, _D : _D + 1]
    o_ref[...] = jnp.where(den > 0.0, num / den, 0.0)


def _combine(partials):
    grid = (_N // _BC,)
    return pl.pallas_call(
        _combine_body,
        grid=grid,
        in_specs=[pl.BlockSpec((_NC, _BC, _DP), lambda i: (0, i, 0))],
        out_specs=pl.BlockSpec((_BC, _D), lambda i: (i, 0)),
        out_shape=jax.ShapeDtypeStruct((_N, _D), jnp.float32),
    )(partials)


def kernel(x, feat, edge_index, W_fc, W_dst):
    z, dstf = _matmuls(x, feat, W_fc, W_dst)
    partials = _edge_pass(z, dstf, edge_index)
    return _combine(partials)


# speedup vs baseline: 1.1620x; 1.0405x over previous
import functools

import jax
import jax.numpy as jnp
from jax import lax
from jax.experimental import pallas as pl
from jax.experimental.pallas import tpu as pltpu
from jax.experimental.pallas import tpu_sc as plsc

_N, _E, _D, _FD = 10000, 320000, 128, 64
_DP = 144
_L = 16
_NC, _NS = 2, 16
_NW = _NC * _NS
_EPW = _E // _NW
_C = 40
_NCHUNK = _EPW // _C
_NP = 10240
_RPT = _NP // _NS


_BM = 2000

_GDN = lax.GatherDimensionNumbers(
    offset_dims=(), collapsed_slice_dims=(0,), start_index_map=(0,)
)


def _rot(v, r):
    perm = ((jnp.arange(_L, dtype=jnp.int32) + r) % _L)[:, None]
    return lax.gather(v, perm, dimension_numbers=_GDN, slice_sizes=(1,),
                      mode=lax.GatherScatterMode.PROMISE_IN_BOUNDS)


def _lane_sum(v):
    for r in (8, 4, 2, 1):
        v = v + _rot(v, r)
    return v


def _mm_body(x_ref, feat_ref, wfc_ref, wdst_ref, z_ref, f_ref):
    z_ref[...] = jnp.dot(
        x_ref[...], wfc_ref[...], preferred_element_type=jnp.float32
    )
    f_ref[...] = jnp.dot(
        feat_ref[...], wdst_ref[...], preferred_element_type=jnp.float32
    )


def _matmuls(x, feat, w_fc, w_dst):
    grid = (_N // _BM,)
    return pl.pallas_call(
        _mm_body,
        grid=grid,
        in_specs=[
            pl.BlockSpec((_BM, _D), lambda i: (i, 0)),
            pl.BlockSpec((_BM, _FD), lambda i: (i, 0)),
            pl.BlockSpec((_D, _D), lambda i: (0, 0)),
            pl.BlockSpec((_FD, _D), lambda i: (0, 0)),
        ],
        out_specs=[
            pl.BlockSpec((_BM, _D), lambda i: (i, 0)),
            pl.BlockSpec((_BM, _D), lambda i: (i, 0)),
        ],
        out_shape=[
            jax.ShapeDtypeStruct((_N, _D), jnp.float32),
            jax.ShapeDtypeStruct((_N, _D), jnp.float32),
        ],
    )(x, feat, w_fc, w_dst)


def _edge_body(z_hbm, f_hbm, ei_hbm, out_hbm,
               acc, sidx, didx, sdix, zrows, frows, srows,
               semz, semf, semi, sems):
    cid = lax.axis_index("c")
    sid = lax.axis_index("s")

    zero = jnp.zeros((_L,), jnp.float32)
    for j in range(_C):
        for k in range(_DP // _L):
            srows[0][j, pl.ds(k * _L, _L)] = zero
    row0 = sid * _RPT
    for t in range(_RPT // _C):
        pltpu.async_copy(srows[0], acc.at[pl.ds(row0 + t * _C, _C)], semz)
    for t in range(_RPT // _C):
        pltpu.make_async_copy(srows[0], acc.at[pl.ds(row0 + t * _C, _C)],
                              semz).wait()
    plsc.subcore_barrier()

    wid = sid * _NC + cid
    base = wid * _EPW

    def idx_copies(c, p):
        off = base + c * _C
        return (
            pltpu.make_async_copy(ei_hbm.at[0, pl.ds(off, _C)], sidx[p], semi),
            pltpu.make_async_copy(ei_hbm.at[1, pl.ds(off, _C)], didx[p], semi),
        )

    def gathers(c, p):
        return (
            pltpu.make_async_copy(z_hbm.at[sidx[p]], zrows[p], semz),
            pltpu.make_async_copy(f_hbm.at[didx[p]], frows[p], semf),
        )

    i0s, i0d = idx_copies(0, 0)
    i0s.start()
    i0d.start()
    i1s, i1d = idx_copies(1, 1)
    i1s.start()
    i1d.start()
    i0s.wait()
    i0d.wait()
    g0z, g0f = gathers(0, 0)
    g0z.start()
    g0f.start()

    def pair(i, carry):
        for p in range(2):
            c = 2 * i + p
            gz, gf = gathers(c, p)
            gz.wait()
            gf.wait()

            @pl.when(c + 1 < _NCHUNK)
            def _issue_next():
                ws, wd = idx_copies(c + 1, 1 - p)
                ws.wait()
                wd.wait()
                nz, nf = gathers(c + 1, 1 - p)
                nz.start()
                nf.start()

            @pl.when(c >= 2)
            def _drain():
                pltpu.make_async_copy(srows[p], acc.at[sdix[p]],
                                      sems[p]).wait()

            for off in (0, _L, _C - _L):
                sdix[p][pl.ds(off, _L)] = didx[p][pl.ds(off, _L)]

            @pl.when(c + 2 < _NCHUNK)
            def _prefetch_idx():
                ns, nd = idx_copies(c + 2, p)
                ns.start()
                nd.start()

            lane0 = jnp.where(
                lax.broadcasted_iota(jnp.int32, (_L,), 0) == 0, 1.0, 0.0
            ).astype(jnp.float32)

            @plsc.parallel_loop(0, _C, 1, unroll=8)
            def edge(j):
                accv = zrows[p][j, pl.ds(0, _L)] * frows[p][j, pl.ds(0, _L)]
                for k in range(1, _D // _L):
                    accv = accv + (zrows[p][j, pl.ds(k * _L, _L)]
                                   * frows[p][j, pl.ds(k * _L, _L)])
                ex = jnp.exp(_lane_sum(accv))
                for k in range(_D // _L):
                    srows[p][j, pl.ds(k * _L, _L)] = (
                        zrows[p][j, pl.ds(k * _L, _L)] * ex)
                srows[p][j, pl.ds(_D, _L)] = ex * lane0

            pltpu.async_copy(srows[p], acc.at[sdix[p]], sems[p], add=True)
        return carry

    lax.fori_loop(0, _NCHUNK // 2, pair, 0)
    pltpu.make_async_copy(srows[0], acc.at[sdix[0]], sems[0]).wait()
    pltpu.make_async_copy(srows[1], acc.at[sdix[1]], sems[1]).wait()
    plsc.subcore_barrier()

    pltpu.sync_copy(acc.at[pl.ds(row0, _RPT)],
                    out_hbm.at[cid, pl.ds(row0, _RPT)])


def _edge_pass(z_pad, dstf, edge_index):
    mesh = plsc.VectorSubcoreMesh(core_axis_name="c", subcore_axis_name="s")
    k = pl.kernel(
        lambda z, f, ei, o, acc, s0, s1, d0, d1, x0, x1, z0, z1, f0, f1,
               sr0, sr1, sz, sf, si, ss0, ss1: _edge_body(
            z, f, ei, o, acc, (s0, s1), (d0, d1), (x0, x1), (z0, z1),
            (f0, f1), (sr0, sr1), sz, sf, si, (ss0, ss1)),
        out_type=jax.ShapeDtypeStruct((_NC, _NP, _DP), jnp.float32),
        mesh=mesh,
        scratch_types=[
            pltpu.VMEM_SHARED((_NP, _DP), jnp.float32),
            pltpu.VMEM((_C,), jnp.int32),
            pltpu.VMEM((_C,), jnp.int32),
            pltpu.VMEM((_C,), jnp.int32),
            pltpu.VMEM((_C,), jnp.int32),
            pltpu.VMEM((_C,), jnp.int32),
            pltpu.VMEM((_C,), jnp.int32),
            pltpu.VMEM((_C, _D), jnp.float32),
            pltpu.VMEM((_C, _D), jnp.float32),
            pltpu.VMEM((_C, _D), jnp.float32),
            pltpu.VMEM((_C, _D), jnp.float32),
            pltpu.VMEM((_C, _DP), jnp.float32),
            pltpu.VMEM((_C, _DP), jnp.float32),
            pltpu.SemaphoreType.DMA,
            pltpu.SemaphoreType.DMA,
            pltpu.SemaphoreType.DMA,
            pltpu.SemaphoreType.DMA,
            pltpu.SemaphoreType.DMA,
        ],
        compiler_params=pltpu.CompilerParams(use_tc_tiling_on_sc=False),
    )
    return k(z_pad, dstf, edge_index)


_RPW = _NP // _NW
_CC = 40


def _combine_sc_body(p_hbm, out_hbm, b0, b1, ob, sem0, sem1):
    cid = lax.axis_index("c")
    sid = lax.axis_index("s")
    wid = sid * _NC + cid
    row0 = wid * _RPW
    nch = jnp.minimum(_RPW, jnp.maximum(0, _N - row0)) // _CC

    zperm = jnp.zeros((_L,), jnp.int32)

    def chunk(t, carry):
        r0 = row0 + t * _CC
        c0 = pltpu.make_async_copy(p_hbm.at[0, pl.ds(r0, _CC)], b0, sem0)
        c1 = pltpu.make_async_copy(p_hbm.at[1, pl.ds(r0, _CC)], b1, sem1)
        c0.start()
        c1.start()
        c0.wait()
        c1.wait()

        @plsc.parallel_loop(0, _CC, 1, unroll=8)
        def row(j):
            den = (b0[j, pl.ds(_D, _L)] + b1[j, pl.ds(_D, _L)])
            den = lax.gather(den, zperm[:, None], dimension_numbers=_GDN,
                             slice_sizes=(1,),
                             mode=lax.GatherScatterMode.PROMISE_IN_BOUNDS)
            den = den + 1e-30
            for k in range(_D // _L):
                s = b0[j, pl.ds(k * _L, _L)] + b1[j, pl.ds(k * _L, _L)]
                ob[j, pl.ds(k * _L, _L)] = s / den

        pltpu.sync_copy(ob, out_hbm.at[pl.ds(r0, _CC)])
        return carry

    lax.fori_loop(0, nch, chunk, 0)


def _combine(partials):
    mesh = plsc.VectorSubcoreMesh(core_axis_name="c", subcore_axis_name="s")
    k = pl.kernel(
        _combine_sc_body,
        out_type=jax.ShapeDtypeStruct((_N, _D), jnp.float32),
        mesh=mesh,
        scratch_types=[
            pltpu.VMEM((_CC, _DP), jnp.float32),
            pltpu.VMEM((_CC, _DP), jnp.float32),
            pltpu.VMEM((_CC, _D), jnp.float32),
            pltpu.SemaphoreType.DMA,
            pltpu.SemaphoreType.DMA,
        ],
        compiler_params=pltpu.CompilerParams(use_tc_tiling_on_sc=False),
    )
    return k(partials)


def kernel(x, feat, edge_index, W_fc, W_dst):
    z, dstf = _matmuls(x, feat, W_fc, W_dst)
    partials = _edge_pass(z, dstf, edge_index)
    return _combine(partials)


# speedup vs baseline: 1.1808x; 1.0161x over previous
import functools

import jax
import jax.numpy as jnp
from jax import lax
from jax.experimental import pallas as pl
from jax.experimental.pallas import tpu as pltpu
from jax.experimental.pallas import tpu_sc as plsc

_N, _E, _D, _FD = 10000, 320000, 128, 64
_DP = 144
_L = 16
_NC, _NS = 2, 16
_NW = _NC * _NS
_EPW = _E // _NW
_C = 40
_NCHUNK = _EPW // _C
_NP = 10240
_RPT = _NP // _NS


_BM = 2000

_GDN = lax.GatherDimensionNumbers(
    offset_dims=(), collapsed_slice_dims=(0,), start_index_map=(0,)
)


def _rot(v, r):
    perm = ((jnp.arange(_L, dtype=jnp.int32) + r) % _L)[:, None]
    return lax.gather(v, perm, dimension_numbers=_GDN, slice_sizes=(1,),
                      mode=lax.GatherScatterMode.PROMISE_IN_BOUNDS)


def _lane_sum(v):
    for r in (8, 4, 2, 1):
        v = v + _rot(v, r)
    return v


def _mm_body(x_ref, feat_ref, wfc_ref, wdst_ref, z_ref, f_ref):
    z_ref[...] = jnp.dot(
        x_ref[...], wfc_ref[...], preferred_element_type=jnp.float32
    )
    f_ref[...] = jnp.dot(
        feat_ref[...], wdst_ref[...], preferred_element_type=jnp.float32
    )


def _matmuls(x, feat, w_fc, w_dst):
    grid = (_N // _BM,)
    return pl.pallas_call(
        _mm_body,
        grid=grid,
        in_specs=[
            pl.BlockSpec((_BM, _D), lambda i: (i, 0)),
            pl.BlockSpec((_BM, _FD), lambda i: (i, 0)),
            pl.BlockSpec((_D, _D), lambda i: (0, 0)),
            pl.BlockSpec((_FD, _D), lambda i: (0, 0)),
        ],
        out_specs=[
            pl.BlockSpec((_BM, _D), lambda i: (i, 0)),
            pl.BlockSpec((_BM, _D), lambda i: (i, 0)),
        ],
        out_shape=[
            jax.ShapeDtypeStruct((_N, _D), jnp.float32),
            jax.ShapeDtypeStruct((_N, _D), jnp.float32),
        ],
    )(x, feat, w_fc, w_dst)


def _edge_body(z_hbm, f_hbm, ei_hbm, out_hbm,
               acc, sidx, didx, sdix, zrows, frows, srows,
               semz, semf, semi, sems):
    cid = lax.axis_index("c")
    sid = lax.axis_index("s")

    zero = jnp.zeros((_L,), jnp.float32)
    for j in range(_C):
        for k in range(_DP // _L):
            srows[0][j, pl.ds(k * _L, _L)] = zero
    row0 = sid * _RPT
    for t in range(_RPT // _C):
        pltpu.async_copy(srows[0], acc.at[pl.ds(row0 + t * _C, _C)], semz)
    for t in range(_RPT // _C):
        pltpu.make_async_copy(srows[0], acc.at[pl.ds(row0 + t * _C, _C)],
                              semz).wait()
    plsc.subcore_barrier()

    wid = sid * _NC + cid
    base = wid * _EPW

    def idx_copies(c, p):
        off = base + c * _C
        return (
            pltpu.make_async_copy(ei_hbm.at[0, pl.ds(off, _C)], sidx[p], semi),
            pltpu.make_async_copy(ei_hbm.at[1, pl.ds(off, _C)], didx[p], semi),
        )

    def gathers(c, p):
        return (
            pltpu.make_async_copy(z_hbm.at[sidx[p]], zrows[p], semz),
            pltpu.make_async_copy(f_hbm.at[didx[p]], frows[p], semf),
        )

    i0s, i0d = idx_copies(0, 0)
    i0s.start()
    i0d.start()
    i1s, i1d = idx_copies(1, 1)
    i1s.start()
    i1d.start()
    i0s.wait()
    i0d.wait()
    g0z, g0f = gathers(0, 0)
    g0z.start()
    g0f.start()

    def pair(i, carry):
        for p in range(2):
            c = 2 * i + p
            gz, gf = gathers(c, p)
            gz.wait()
            gf.wait()

            @pl.when(c + 1 < _NCHUNK)
            def _issue_next():
                ws, wd = idx_copies(c + 1, 1 - p)
                ws.wait()
                wd.wait()
                nz, nf = gathers(c + 1, 1 - p)
                nz.start()
                nf.start()

            @pl.when(c >= 2)
            def _drain():
                pltpu.make_async_copy(srows[p], acc.at[sdix[p]],
                                      sems[p]).wait()

            for off in (0, _L, _C - _L):
                sdix[p][pl.ds(off, _L)] = didx[p][pl.ds(off, _L)]

            @pl.when(c + 2 < _NCHUNK)
            def _prefetch_idx():
                ns, nd = idx_copies(c + 2, p)
                ns.start()
                nd.start()

            lane0 = jnp.where(
                lax.broadcasted_iota(jnp.int32, (_L,), 0) == 0, 1.0, 0.0
            ).astype(jnp.float32)

            @plsc.parallel_loop(0, _C, 1, unroll=8)
            def edge(j):
                accv = zrows[p][j, pl.ds(0, _L)] * frows[p][j, pl.ds(0, _L)]
                for k in range(1, _D // _L):
                    accv = accv + (zrows[p][j, pl.ds(k * _L, _L)]
                                   * frows[p][j, pl.ds(k * _L, _L)])
                ex = jnp.exp(_lane_sum(accv))
                for k in range(_D // _L):
                    srows[p][j, pl.ds(k * _L, _L)] = (
                        zrows[p][j, pl.ds(k * _L, _L)] * ex)
                srows[p][j, pl.ds(_D, _L)] = ex * lane0

            pltpu.async_copy(srows[p], acc.at[sdix[p]], sems[p], add=True)
        return carry

    lax.fori_loop(0, _NCHUNK // 2, pair, 0)
    pltpu.make_async_copy(srows[0], acc.at[sdix[0]], sems[0]).wait()
    pltpu.make_async_copy(srows[1], acc.at[sdix[1]], sems[1]).wait()
    plsc.subcore_barrier()

    pltpu.sync_copy(acc.at[pl.ds(row0, _RPT)],
                    out_hbm.at[cid, pl.ds(row0, _RPT)])


def _edge_pass(z_pad, dstf, edge_index):
    mesh = plsc.VectorSubcoreMesh(core_axis_name="c", subcore_axis_name="s")
    k = pl.kernel(
        lambda z, f, ei, o, acc, s0, s1, d0, d1, x0, x1, z0, z1, f0, f1,
               sr0, sr1, sz, sf, si, ss0, ss1: _edge_body(
            z, f, ei, o, acc, (s0, s1), (d0, d1), (x0, x1), (z0, z1),
            (f0, f1), (sr0, sr1), sz, sf, si, (ss0, ss1)),
        out_type=jax.ShapeDtypeStruct((_NC, _NP, _DP), jnp.float32),
        mesh=mesh,
        scratch_types=[
            pltpu.VMEM_SHARED((_NP, _DP), jnp.float32),
            pltpu.VMEM((_C,), jnp.int32),
            pltpu.VMEM((_C,), jnp.int32),
            pltpu.VMEM((_C,), jnp.int32),
            pltpu.VMEM((_C,), jnp.int32),
            pltpu.VMEM((_C,), jnp.int32),
            pltpu.VMEM((_C,), jnp.int32),
            pltpu.VMEM((_C, _D), jnp.float32),
            pltpu.VMEM((_C, _D), jnp.float32),
            pltpu.VMEM((_C, _D), jnp.float32),
            pltpu.VMEM((_C, _D), jnp.float32),
            pltpu.VMEM((_C, _DP), jnp.float32),
            pltpu.VMEM((_C, _DP), jnp.float32),
            pltpu.SemaphoreType.DMA,
            pltpu.SemaphoreType.DMA,
            pltpu.SemaphoreType.DMA,
            pltpu.SemaphoreType.DMA,
            pltpu.SemaphoreType.DMA,
        ],
        compiler_params=pltpu.CompilerParams(use_tc_tiling_on_sc=False),
    )
    return k(z_pad, dstf, edge_index)


_RPW = _NP // _NW
_CC = 40


def _combine_sc_body(p_hbm, out_hbm, b0, b1, ob, sem0, sem1):
    cid = lax.axis_index("c")
    sid = lax.axis_index("s")
    wid = sid * _NC + cid
    row0 = wid * _RPW
    nch = jnp.minimum(_RPW, jnp.maximum(0, _N - row0)) // _CC

    zperm = jnp.zeros((_L,), jnp.int32)

    def loads(t, q):
        r0 = row0 + t * _CC
        return (
            pltpu.make_async_copy(p_hbm.at[0, pl.ds(r0, _CC)], b0[q], sem0),
            pltpu.make_async_copy(p_hbm.at[1, pl.ds(r0, _CC)], b1[q], sem1),
        )

    l0a, l0b = loads(0, 0)
    l0a.start()
    l0b.start()

    def pairloop(i, carry):
        for q in range(2):
            t = 2 * i + q
            ca, cb = loads(t, q)
            ca.wait()
            cb.wait()

            @pl.when(t + 1 < nch)
            def _prefetch():
                na, nb = loads(t + 1, 1 - q)
                na.start()
                nb.start()

            @plsc.parallel_loop(0, _CC, 1, unroll=8)
            def row(j):
                den = (b0[q][j, pl.ds(_D, _L)] + b1[q][j, pl.ds(_D, _L)])
                den = lax.gather(den, zperm[:, None], dimension_numbers=_GDN,
                                 slice_sizes=(1,),
                                 mode=lax.GatherScatterMode.PROMISE_IN_BOUNDS)
                den = den + 1e-30
                for k in range(_D // _L):
                    s = (b0[q][j, pl.ds(k * _L, _L)]
                         + b1[q][j, pl.ds(k * _L, _L)])
                    ob[j, pl.ds(k * _L, _L)] = s / den

            pltpu.sync_copy(ob, out_hbm.at[pl.ds(row0 + t * _CC, _CC)])
        return carry

    lax.fori_loop(0, nch // 2, pairloop, 0)


def _combine(partials):
    mesh = plsc.VectorSubcoreMesh(core_axis_name="c", subcore_axis_name="s")
    k = pl.kernel(
        lambda p, o, b00, b01, b10, b11, ob, s0, s1: _combine_sc_body(
            p, o, (b00, b01), (b10, b11), ob, s0, s1),
        out_type=jax.ShapeDtypeStruct((_N, _D), jnp.float32),
        mesh=mesh,
        scratch_types=[
            pltpu.VMEM((_CC, _DP), jnp.float32),
            pltpu.VMEM((_CC, _DP), jnp.float32),
            pltpu.VMEM((_CC, _DP), jnp.float32),
            pltpu.VMEM((_CC, _DP), jnp.float32),
            pltpu.VMEM((_CC, _D), jnp.float32),
            pltpu.SemaphoreType.DMA,
            pltpu.SemaphoreType.DMA,
        ],
        compiler_params=pltpu.CompilerParams(use_tc_tiling_on_sc=False),
    )
    return k(partials)


def kernel(x, feat, edge_index, W_fc, W_dst):
    z, dstf = _matmuls(x, feat, W_fc, W_dst)
    partials = _edge_pass(z, dstf, edge_index)
    return _combine(partials)
